# baseline (speedup 1.0000x reference)
import jax
import jax.numpy as jnp
from jax.experimental import pallas as pl
from jax.experimental.pallas import tpu as pltpu

_NUM_SEGMENTS = 16
_BLK = 2048
_SUB = 256


def _body(bounds_ref, seg_ref, x_ref, w_ref, b_ref, out_ref):
    i = pl.program_id(0)
    nsub = _BLK // _SUB

    @pl.when(i == 0)
    def _init():
        out_ref[...] = jnp.full_like(out_ref, -jnp.inf)

    wv = w_ref[...].astype(jnp.bfloat16)
    for t in range(nsub):
        xt = x_ref[pl.ds(t * _SUB, _SUB), :].astype(jnp.bfloat16)
        d = jnp.dot(xt, wv, preferred_element_type=jnp.float32)
        segt = seg_ref[pl.ds(t * _SUB, _SUB), :]
        jj = i * nsub + t
        lo = bounds_ref[2 * jj]
        hi = bounds_ref[2 * jj + 1]

        def _fold(s, carry):
            m = jnp.max(jnp.where(segt == s, d, -jnp.inf), axis=0, keepdims=True)
            out_ref[pl.ds(s, 1), :] = jnp.maximum(out_ref[pl.ds(s, 1), :], m)
            return carry

        jax.lax.fori_loop(lo, hi + 1, _fold, 0)

    @pl.when(i == pl.num_programs(0) - 1)
    def _finish():
        out_ref[...] = jnp.maximum(out_ref[...], -b_ref[...]) + b_ref[...]


def kernel(x, W, b, batch):
    n, d_in = x.shape
    d_out = W.shape[1]
    g = n // _BLK
    seg = batch.astype(jnp.int32)
    bounds = jnp.stack([seg[::_SUB], seg[_SUB - 1 :: _SUB]], axis=1).reshape(-1)
    seg2d = seg.reshape(n, 1)

    grid_spec = pltpu.PrefetchScalarGridSpec(
        num_scalar_prefetch=1,
        grid=(g,),
        in_specs=[
            pl.BlockSpec((_BLK, 1), lambda i, *_: (i, 0)),
            pl.BlockSpec((_BLK, d_in), lambda i, *_: (i, 0)),
            pl.BlockSpec((d_in, d_out), lambda i, *_: (0, 0)),
            pl.BlockSpec((1, d_out), lambda i, *_: (0, 0)),
        ],
        out_specs=pl.BlockSpec((_NUM_SEGMENTS, d_out), lambda i, *_: (0, 0)),
    )
    out = pl.pallas_call(
        _body,
        grid_spec=grid_spec,
        out_shape=jax.ShapeDtypeStruct((_NUM_SEGMENTS, d_out), jnp.float32),
        compiler_params=pltpu.CompilerParams(
            dimension_semantics=("arbitrary",),
        ),
    )(bounds, seg2d, x, W, b.reshape(1, d_out))

    new_batch = jnp.arange(_NUM_SEGMENTS, dtype=jnp.int64)
    return (out, new_batch)


# speedup vs baseline: 1.2380x; 1.2380x over previous
import jax
import jax.numpy as jnp
from jax.experimental import pallas as pl
from jax.experimental.pallas import tpu as pltpu

_NUM_SEGMENTS = 16
_BLK = 2048
_SUB = 256


def _body(bounds_ref, seg_ref, x_ref, w_ref, b_ref, out_ref):
    i = pl.program_id(0)
    nsub = _BLK // _SUB

    @pl.when(i == 0)
    def _init():
        out_ref[...] = jnp.full_like(out_ref, -jnp.inf)

    wv = w_ref[...]
    for t in range(nsub):
        xt = x_ref[pl.ds(t * _SUB, _SUB), :].astype(jnp.bfloat16)
        d = jnp.dot(xt, wv, preferred_element_type=jnp.float32)
        segt = seg_ref[pl.ds(t * _SUB, _SUB), :]
        jj = i * nsub + t
        lo = bounds_ref[2 * jj]
        hi = bounds_ref[2 * jj + 1]

        m_lo = jnp.max(jnp.where(segt == lo, d, -jnp.inf), axis=0, keepdims=True)
        out_ref[pl.ds(lo, 1), :] = jnp.maximum(out_ref[pl.ds(lo, 1), :], m_lo)
        m_hi = jnp.max(jnp.where(segt == hi, d, -jnp.inf), axis=0, keepdims=True)
        out_ref[pl.ds(hi, 1), :] = jnp.maximum(out_ref[pl.ds(hi, 1), :], m_hi)

        @pl.when(hi - lo >= 2)
        def _interior():
            def _fold(s, carry):
                m = jnp.max(jnp.where(segt == s, d, -jnp.inf), axis=0, keepdims=True)
                out_ref[pl.ds(s, 1), :] = jnp.maximum(out_ref[pl.ds(s, 1), :], m)
                return carry

            jax.lax.fori_loop(lo + 1, hi, _fold, 0)

    @pl.when(i == pl.num_programs(0) - 1)
    def _finish():
        out_ref[...] = jnp.maximum(out_ref[...], -b_ref[...]) + b_ref[...]


def kernel(x, W, b, batch):
    n, d_in = x.shape
    d_out = W.shape[1]
    g = n // _BLK
    seg = batch.astype(jnp.int32)
    bounds = jnp.stack([seg[::_SUB], seg[_SUB - 1 :: _SUB]], axis=1).reshape(-1)
    seg2d = seg.reshape(n, 1)

    grid_spec = pltpu.PrefetchScalarGridSpec(
        num_scalar_prefetch=1,
        grid=(g,),
        in_specs=[
            pl.BlockSpec((_BLK, 1), lambda i, *_: (i, 0)),
            pl.BlockSpec((_BLK, d_in), lambda i, *_: (i, 0)),
            pl.BlockSpec((d_in, d_out), lambda i, *_: (0, 0)),
            pl.BlockSpec((1, d_out), lambda i, *_: (0, 0)),
        ],
        out_specs=pl.BlockSpec((_NUM_SEGMENTS, d_out), lambda i, *_: (0, 0)),
    )
    out = pl.pallas_call(
        _body,
        grid_spec=grid_spec,
        out_shape=jax.ShapeDtypeStruct((_NUM_SEGMENTS, d_out), jnp.float32),
        compiler_params=pltpu.CompilerParams(
            dimension_semantics=("arbitrary",),
        ),
    )(bounds, seg2d, x, W.astype(jnp.bfloat16), b.reshape(1, d_out))

    new_batch = jnp.arange(_NUM_SEGMENTS, dtype=jnp.int64)
    return (out, new_batch)
